# baseline (speedup 1.0000x reference)
import functools

import jax
import jax.numpy as jnp
from jax.experimental import pallas as pl
from jax.experimental.pallas import tpu as pltpu

B, T, D = 16, 4096, 128
CH = 1024
NCH = T // CH
NEG_INF = float("-inf")


def _body(lens_ref, x_hbm, wt_ref, b_ref, out_ref, xbuf, sems):
    i = pl.program_id(0)
    length = lens_ref[i]

    for k in range(NCH):
        @pl.when(k * CH < length)
        def _start():
            pltpu.make_async_copy(
                x_hbm.at[i, pl.ds(k * CH, CH), :],
                xbuf.at[k],
                sems.at[k],
            ).start()

    def _dot(x):
        return (
            jnp.dot(x, wt_ref[...], preferred_element_type=jnp.float32)
            + b_ref[...]
        )

    for k in range(NCH):
        t0 = k * CH

        @pl.when(t0 + CH <= length)
        def _full_valid():
            pltpu.make_async_copy(
                x_hbm.at[i, pl.ds(t0, CH), :], xbuf.at[k], sems.at[k]
            ).wait()
            out_ref[0, pl.ds(t0, CH), :] = _dot(xbuf[k])

        @pl.when(t0 >= length)
        def _full_pad():
            out_ref[0, pl.ds(t0, CH), :] = jnp.full(
                (CH, D), NEG_INF, dtype=jnp.float32
            )

        @pl.when(jnp.logical_and(t0 < length, t0 + CH > length))
        def _partial():
            pltpu.make_async_copy(
                x_hbm.at[i, pl.ds(t0, CH), :], xbuf.at[k], sems.at[k]
            ).wait()
            rows = t0 + jax.lax.broadcasted_iota(jnp.int32, (CH, D), 0)
            out_ref[0, pl.ds(t0, CH), :] = jnp.where(
                rows < length, _dot(xbuf[k]), NEG_INF
            )


@functools.partial(jax.jit, static_argnames=())
def _run(padded, lengths, wt, b2):
    grid_spec = pltpu.PrefetchScalarGridSpec(
        num_scalar_prefetch=1,
        grid=(B,),
        in_specs=[
            pl.BlockSpec(memory_space=pl.ANY),
            pl.BlockSpec((D, D), lambda i, lens: (0, 0)),
            pl.BlockSpec((1, D), lambda i, lens: (0, 0)),
        ],
        out_specs=pl.BlockSpec((1, T, D), lambda i, lens: (i, 0, 0)),
        scratch_shapes=[
            pltpu.VMEM((NCH, CH, D), jnp.float32),
            pltpu.SemaphoreType.DMA((NCH,)),
        ],
    )
    out = pl.pallas_call(
        _body,
        grid_spec=grid_spec,
        out_shape=jax.ShapeDtypeStruct((B, T, D), jnp.float32),
        compiler_params=pltpu.CompilerParams(
            dimension_semantics=("arbitrary",),
        ),
    )(lengths, padded, wt, b2)
    return out


def kernel(padded, lengths, W, b):
    wt = W.T
    b2 = b.reshape(1, D)
    out = _run(padded, lengths.astype(jnp.int32), wt, b2)
    return out, lengths


# speedup vs baseline: 1.5323x; 1.5323x over previous
import functools

import jax
import jax.numpy as jnp
from jax.experimental import pallas as pl
from jax.experimental.pallas import tpu as pltpu

B, T, D = 16, 4096, 128
CH = 1024
NCH = T // CH
NEG_INF = float("-inf")


def _body(lens_ref, x_hbm, wt_ref, b_ref, out_ref, xbuf, sems):
    i = pl.program_id(0)
    length = lens_ref[i]

    def _start_row(row, slot):
        row_len = lens_ref[row]
        for k in range(NCH):
            @pl.when(k * CH < row_len)
            def _start():
                pltpu.make_async_copy(
                    x_hbm.at[row, pl.ds(k * CH, CH), :],
                    xbuf.at[slot, k],
                    sems.at[slot, k],
                ).start()

    @pl.when(i == 0)
    def _prologue():
        _start_row(0, 0)

    @pl.when(i + 1 < B)
    def _prefetch_next():
        _start_row(i + 1, (i + 1) % 2)

    slot = i % 2

    def _dot(x):
        return (
            jnp.dot(x, wt_ref[...], preferred_element_type=jnp.float32)
            + b_ref[...]
        )

    def _wait(k):
        pltpu.make_async_copy(
            x_hbm.at[i, pl.ds(k * CH, CH), :],
            xbuf.at[slot, k],
            sems.at[slot, k],
        ).wait()

    for k in range(NCH):
        t0 = k * CH

        @pl.when(t0 + CH <= length)
        def _full_valid():
            _wait(k)
            out_ref[0, pl.ds(t0, CH), :] = _dot(xbuf[slot, k])

        @pl.when(t0 >= length)
        def _full_pad():
            out_ref[0, pl.ds(t0, CH), :] = jnp.full(
                (CH, D), NEG_INF, dtype=jnp.float32
            )

        @pl.when(jnp.logical_and(t0 < length, t0 + CH > length))
        def _partial():
            _wait(k)
            rows = t0 + jax.lax.broadcasted_iota(jnp.int32, (CH, D), 0)
            out_ref[0, pl.ds(t0, CH), :] = jnp.where(
                rows < length, _dot(xbuf[slot, k]), NEG_INF
            )


@functools.partial(jax.jit, static_argnames=())
def _run(padded, lengths, wt, b2):
    grid_spec = pltpu.PrefetchScalarGridSpec(
        num_scalar_prefetch=1,
        grid=(B,),
        in_specs=[
            pl.BlockSpec(memory_space=pl.ANY),
            pl.BlockSpec((D, D), lambda i, lens: (0, 0)),
            pl.BlockSpec((1, D), lambda i, lens: (0, 0)),
        ],
        out_specs=pl.BlockSpec((1, T, D), lambda i, lens: (i, 0, 0)),
        scratch_shapes=[
            pltpu.VMEM((2, NCH, CH, D), jnp.float32),
            pltpu.SemaphoreType.DMA((2, NCH)),
        ],
    )
    out = pl.pallas_call(
        _body,
        grid_spec=grid_spec,
        out_shape=jax.ShapeDtypeStruct((B, T, D), jnp.float32),
        compiler_params=pltpu.CompilerParams(
            dimension_semantics=("arbitrary",),
        ),
    )(lengths, padded, wt, b2)
    return out


def kernel(padded, lengths, W, b):
    wt = W.T
    b2 = b.reshape(1, D)
    out = _run(padded, lengths.astype(jnp.int32), wt, b2)
    return out, lengths
